# Initial kernel scaffold; baseline (speedup 1.0000x reference)
import jax
import jax.numpy as jnp
from jax.experimental import pallas as pl


def kernel(features, idx7, idx3, p7, p3, out_params):
    raise NotImplementedError("write your pallas kernel here")


# speedup vs baseline: 7.2208x; 7.2208x over previous
import functools

import jax
import jax.numpy as jnp
from jax.experimental import pallas as pl
from jax.experimental.pallas import tpu as pltpu

B = 2048
EPS = 1e-5
H = 16
N_TREES = 100
NP = 256
NC = 8
BT1 = 256
TT = 128
NH = NP // 2


def _fold_bn(g, b, m, v):
    a = g * jax.lax.rsqrt(v + EPS)
    return a, b - m * a


def _pad_n(x, n_to):
    pad = [(0, n_to - x.shape[0])] + [(0, 0)] * (x.ndim - 1)
    return jnp.pad(x, pad)


def _prep(idx7, idx3, p7, p3):
    n7, n3 = idx7.shape[0], idx3.shape[0]
    n = n7 + n3

    a7, c7 = _fold_bn(p7['bn0_g'], p7['bn0_b'], p7['bn0_m'], p7['bn0_v'])
    a3, c3 = _fold_bn(p3['bn0_g'], p3['bn0_b'], p3['bn0_m'], p3['bn0_v'])
    a3 = jnp.pad(a3, ((0, 0), (0, 4)))
    c3 = jnp.pad(c3, ((0, 0), (0, 4)))
    w0_3 = jnp.pad(p3['W0'], ((0, 0), (0, 4), (0, 0)))
    idx3p = jnp.pad(idx3, ((0, 0), (0, 4)))
    a_all = _pad_n(jnp.concatenate([a7, a3], 0), NP)
    c_all = _pad_n(jnp.concatenate([c7, c3], 0), NP)
    w0_all = _pad_n(jnp.concatenate([p7['W0'], w0_3], 0), NP)
    cols = _pad_n(jnp.concatenate([idx7, idx3p], 0), NP)
    upd = (a_all[..., None] * w0_all).reshape(-1, 32)
    nn = jnp.repeat(jnp.arange(NP), 7)
    M = jnp.zeros((19, NP, 32), jnp.float32).at[cols.reshape(-1), nn].add(upd)
    M = M.reshape(19, NP * 32)
    b0_all = _pad_n(jnp.concatenate([p7['b0'], p3['b0']], 0), NP)
    B0 = b0_all + (c_all[..., None] * w0_all).sum(1)
    B0r = B0.reshape(NP // NC, 1, NC * 32)

    def cat(name):
        return _pad_n(jnp.concatenate([p7[name], p3[name]], 0), NP)

    a1, c1 = _fold_bn(cat('bn1_g'), cat('bn1_b'), cat('bn1_m'), cat('bn1_v'))
    W1e = cat('W1') * a1[:, None, :]
    B1e = cat('b1') * a1 + c1
    a2, c2 = _fold_bn(cat('bn2_g'), cat('bn2_b'), cat('bn2_m'), cat('bn2_v'))
    W2e = cat('W2') * a2[:, None, :]
    B2e = cat('b2') * a2 + c2

    Wgf = jnp.transpose(cat('Wih_f'), (0, 2, 1))
    Wgb = jnp.transpose(cat('Wih_b'), (0, 2, 1))
    Bgf, Bgb = cat('bih_f'), cat('bih_b')

    Whf = jnp.transpose(cat('Whh_f'), (1, 2, 0))
    Whb = jnp.transpose(cat('Whh_b'), (1, 2, 0))
    bhf = jnp.transpose(cat('bhh_f'), (1, 0))
    bhb = jnp.transpose(cat('bhh_b'), (1, 0))

    wout = cat('Wout')
    Wof = jnp.transpose(wout[:, :H, 0], (1, 0))
    Wob = jnp.transpose(wout[:, H:, 0], (1, 0))
    bo = cat('bout').reshape(1, NP)
    return (M, B0r, W1e, B1e, W2e, B2e, Wgf, Bgf, Wgb, Bgb,
            Whf, Whb, bhf, bhb, Wof, Wob, bo)


def _k1_body(f_ref, m_ref, b0_ref, w1_ref, b1_ref, w2_ref, b2_ref,
             wgf_ref, bgf_ref, wgb_ref, bgb_ref, gif_ref, gib_ref):
    feat = f_ref[...]
    cls = jnp.clip(feat[:, 0].astype(jnp.int32), 0, 4)
    iota = jax.lax.broadcasted_iota(jnp.int32, (BT1, 5), 1)
    oh = (cls[:, None] == iota).astype(jnp.float32)
    f19 = jnp.concatenate([oh, feat], axis=1)
    h0 = jnp.maximum(
        jnp.dot(f19, m_ref[...], preferred_element_type=jnp.float32)
        + b0_ref[0], 0.0)
    for j in range(NC):
        h0j = h0[:, j * 32:(j + 1) * 32]
        h1 = jnp.maximum(
            jnp.dot(h0j, w1_ref[j], preferred_element_type=jnp.float32)
            + b1_ref[j][None, :], 0.0)
        h2 = jnp.maximum(
            jnp.dot(h1, w2_ref[j], preferred_element_type=jnp.float32)
            + b2_ref[j][None, :], 0.0)
        gf = (jnp.dot(h2, wgf_ref[j], preferred_element_type=jnp.float32)
              + bgf_ref[j][None, :])
        gb = (jnp.dot(h2, wgb_ref[j], preferred_element_type=jnp.float32)
              + bgb_ref[j][None, :])
        gif_ref[:, j, :] = gf
        gib_ref[:, j, :] = gb


def _run_k1(features, M, B0r, W1e, B1e, W2e, B2e, Wgf, Bgf, Wgb, Bgb):
    grid = (NP // NC, B // BT1)
    wspec = lambda bs: pl.BlockSpec(bs, lambda c, b: (c, 0, 0))
    bspec = lambda bs: pl.BlockSpec(bs, lambda c, b: (c, 0))
    return pl.pallas_call(
        _k1_body,
        grid=grid,
        in_specs=[
            pl.BlockSpec((BT1, 14), lambda c, b: (b, 0)),
            pl.BlockSpec((19, NC * 32), lambda c, b: (0, c)),
            pl.BlockSpec((1, 1, NC * 32), lambda c, b: (c, 0, 0)),
            wspec((NC, 32, 64)), bspec((NC, 64)),
            wspec((NC, 64, 32)), bspec((NC, 32)),
            wspec((NC, 32, 48)), bspec((NC, 48)),
            wspec((NC, 32, 48)), bspec((NC, 48)),
        ],
        out_specs=[
            pl.BlockSpec((BT1, NC, 48), lambda c, b: (b, c, 0)),
            pl.BlockSpec((BT1, NC, 48), lambda c, b: (b, c, 0)),
        ],
        out_shape=[
            jax.ShapeDtypeStruct((B, NP, 48), jnp.float32),
            jax.ShapeDtypeStruct((B, NP, 48), jnp.float32),
        ],
        compiler_params=pltpu.CompilerParams(
            dimension_semantics=("parallel", "parallel")),
    )(features, M, B0r, W1e, B1e, W2e, B2e, Wgf, Bgf, Wgb, Bgb)


def _gru_step(gi, gh, h):
    r = jax.nn.sigmoid(gi[0:H] + gh[0:H])
    z = jax.nn.sigmoid(gi[H:2 * H] + gh[H:2 * H])
    c = jnp.tanh(gi[2 * H:] + r * gh[2 * H:])
    return (1.0 - z) * c + z * h


def _k2_body(gitf_ref, gitb_ref, whf_ref, whb_ref, bhf_ref, bhb_ref,
             wof_ref, wob_ref, of_ref, ob_ref, hf_scr, hb_scr):
    k = pl.program_id(1)

    @pl.when(k == 0)
    def _():
        hf_scr[...] = jnp.zeros_like(hf_scr)
        hb_scr[...] = jnp.zeros_like(hb_scr)

    whf, whb = whf_ref[...], whb_ref[...]
    bhf, bhb = bhf_ref[...], bhb_ref[...]
    wof, wob = wof_ref[...], wob_ref[...]

    def body(s, carry):
        h_f, h_b = carry
        gif = gitf_ref[s]
        ghf = (whf * h_f[None, :, :]).sum(axis=1) + bhf
        h_f = _gru_step(gif, ghf, h_f)
        of_ref[pl.ds(s, 1), :] = (h_f * wof).sum(0, keepdims=True)
        sb = TT - 1 - s
        gib = gitb_ref[sb]
        ghb = (whb * h_b[None, :, :]).sum(axis=1) + bhb
        h_b = _gru_step(gib, ghb, h_b)
        ob_ref[pl.ds(sb, 1), :] = (h_b * wob).sum(0, keepdims=True)
        return h_f, h_b

    hf, hb = jax.lax.fori_loop(0, TT, body, (hf_scr[...], hb_scr[...]))
    hf_scr[...] = hf
    hb_scr[...] = hb


def _run_k2(gitf, gitb, Whf, Whb, bhf, bhb, Wof, Wob):
    nt = B // TT
    grid = (2, nt)
    half3 = lambda: pl.BlockSpec((48, H, NH), lambda c, k: (0, 0, c))
    half2 = lambda a, b_: pl.BlockSpec((a, b_), lambda c, k: (0, c))
    return pl.pallas_call(
        _k2_body,
        grid=grid,
        in_specs=[
            pl.BlockSpec((TT, 48, NH), lambda c, k: (k, 0, c)),
            pl.BlockSpec((TT, 48, NH), lambda c, k: (nt - 1 - k, 0, c)),
            half3(), half3(),
            half2(48, NH), half2(48, NH),
            half2(H, NH), half2(H, NH),
        ],
        out_specs=[
            pl.BlockSpec((TT, NH), lambda c, k: (k, c)),
            pl.BlockSpec((TT, NH), lambda c, k: (nt - 1 - k, c)),
        ],
        out_shape=[
            jax.ShapeDtypeStruct((B, NP), jnp.float32),
            jax.ShapeDtypeStruct((B, NP), jnp.float32),
        ],
        scratch_shapes=[
            pltpu.VMEM((H, NH), jnp.float32),
            pltpu.VMEM((H, NH), jnp.float32),
        ],
        compiler_params=pltpu.CompilerParams(
            dimension_semantics=("parallel", "arbitrary")),
    )(gitf, gitb, Whf, Whb, bhf, bhb, Wof, Wob)


def _k3_body(of_ref, ob_ref, bo_ref, w1_ref, b1_ref, w2_ref, b2_ref,
             dl_ref, res_ref):
    dl = jax.nn.sigmoid(of_ref[...] + ob_ref[...] + bo_ref[...])
    dl_ref[...] = dl
    hid = jnp.maximum(
        jnp.dot(dl, w1_ref[...], preferred_element_type=jnp.float32)
        + b1_ref[...], 0.0)
    res_ref[...] = jax.nn.sigmoid(
        (hid * w2_ref[...]).sum(axis=1, keepdims=True) + b2_ref[...])


def _run_k3(of, ob, bo, W1p, b1, W2r, b2):
    grid = (B // BT1,)
    fixed = lambda bs: pl.BlockSpec(bs, lambda b: tuple(0 for _ in bs))
    return pl.pallas_call(
        _k3_body,
        grid=grid,
        in_specs=[
            pl.BlockSpec((BT1, NP), lambda b: (b, 0)),
            pl.BlockSpec((BT1, NP), lambda b: (b, 0)),
            fixed((1, NP)), fixed((NP, 16)), fixed((1, 16)),
            fixed((1, 16)), fixed((1, 1)),
        ],
        out_specs=[
            pl.BlockSpec((BT1, NP), lambda b: (b, 0)),
            pl.BlockSpec((BT1, 1), lambda b: (b, 0)),
        ],
        out_shape=[
            jax.ShapeDtypeStruct((B, NP), jnp.float32),
            jax.ShapeDtypeStruct((B, 1), jnp.float32),
        ],
        compiler_params=pltpu.CompilerParams(
            dimension_semantics=("parallel",)),
    )(of, ob, bo, W1p, b1, W2r, b2)


@functools.partial(jax.jit, static_argnames=())
def _forward(features, idx7, idx3, p7, p3, out_params):
    (M, B0r, W1e, B1e, W2e, B2e, Wgf, Bgf, Wgb, Bgb,
     Whf, Whb, bhf, bhb, Wof, Wob, bo) = _prep(idx7, idx3, p7, p3)

    gif, gib = _run_k1(features, M, B0r, W1e, B1e, W2e, B2e,
                       Wgf, Bgf, Wgb, Bgb)
    gitf = jnp.transpose(gif, (0, 2, 1))
    gitb = jnp.transpose(gib, (0, 2, 1))
    of, ob = _run_k2(gitf, gitb, Whf, Whb, bhf, bhb, Wof, Wob)

    n = idx7.shape[0] + idx3.shape[0]
    W1p = jnp.zeros((NP, 16), jnp.float32).at[:n].set(out_params['W1'][:n])
    b1 = out_params['b1'].reshape(1, 16)
    W2r = out_params['W2'].reshape(1, 16)
    b2 = out_params['b2'].reshape(1, 1)
    dl, res = _run_k3(of, ob, bo, W1p, b1, W2r, b2)
    return res, dl[:, :n]


def kernel(features, idx7, idx3, p7, p3, out_params):
    return _forward(features, idx7, idx3, p7, p3, out_params)


# speedup vs baseline: 7.3061x; 1.0118x over previous
import functools

import jax
import jax.numpy as jnp
from jax.experimental import pallas as pl
from jax.experimental.pallas import tpu as pltpu

B = 2048
EPS = 1e-5
H = 16
N_TREES = 100
NP = 256
NC = 8
BT1 = 256
TT = 128
NH = NP // 2


def _fold_bn(g, b, m, v):
    a = g * jax.lax.rsqrt(v + EPS)
    return a, b - m * a


def _pad_n(x, n_to):
    pad = [(0, n_to - x.shape[0])] + [(0, 0)] * (x.ndim - 1)
    return jnp.pad(x, pad)


def _prep(idx7, idx3, p7, p3):
    n7, n3 = idx7.shape[0], idx3.shape[0]
    n = n7 + n3

    a7, c7 = _fold_bn(p7['bn0_g'], p7['bn0_b'], p7['bn0_m'], p7['bn0_v'])
    a3, c3 = _fold_bn(p3['bn0_g'], p3['bn0_b'], p3['bn0_m'], p3['bn0_v'])
    a3 = jnp.pad(a3, ((0, 0), (0, 4)))
    c3 = jnp.pad(c3, ((0, 0), (0, 4)))
    w0_3 = jnp.pad(p3['W0'], ((0, 0), (0, 4), (0, 0)))
    idx3p = jnp.pad(idx3, ((0, 0), (0, 4)))
    a_all = _pad_n(jnp.concatenate([a7, a3], 0), NP)
    c_all = _pad_n(jnp.concatenate([c7, c3], 0), NP)
    w0_all = _pad_n(jnp.concatenate([p7['W0'], w0_3], 0), NP)
    cols = _pad_n(jnp.concatenate([idx7, idx3p], 0), NP)
    upd = (a_all[..., None] * w0_all).reshape(-1, 32)
    nn = jnp.repeat(jnp.arange(NP), 7)
    M = jnp.zeros((19, NP, 32), jnp.float32).at[cols.reshape(-1), nn].add(upd)
    M = M.reshape(19, NP * 32)
    b0_all = _pad_n(jnp.concatenate([p7['b0'], p3['b0']], 0), NP)
    B0 = b0_all + (c_all[..., None] * w0_all).sum(1)
    B0r = B0.reshape(NP // NC, 1, NC * 32)

    def cat(name):
        return _pad_n(jnp.concatenate([p7[name], p3[name]], 0), NP)

    a1, c1 = _fold_bn(cat('bn1_g'), cat('bn1_b'), cat('bn1_m'), cat('bn1_v'))
    W1e = cat('W1') * a1[:, None, :]
    B1e = cat('b1') * a1 + c1
    a2, c2 = _fold_bn(cat('bn2_g'), cat('bn2_b'), cat('bn2_m'), cat('bn2_v'))
    W2e = cat('W2') * a2[:, None, :]
    B2e = cat('b2') * a2 + c2

    Wgf = jnp.transpose(cat('Wih_f'), (0, 2, 1))
    Wgb = jnp.transpose(cat('Wih_b'), (0, 2, 1))
    Bgf, Bgb = cat('bih_f'), cat('bih_b')

    Whf = jnp.transpose(cat('Whh_f'), (1, 2, 0))
    Whb = jnp.transpose(cat('Whh_b'), (1, 2, 0))
    bhf = jnp.transpose(cat('bhh_f'), (1, 0))
    bhb = jnp.transpose(cat('bhh_b'), (1, 0))

    wout = cat('Wout')
    Wof = jnp.transpose(wout[:, :H, 0], (1, 0))
    Wob = jnp.transpose(wout[:, H:, 0], (1, 0))
    bo = cat('bout').reshape(1, NP)
    return (M, B0r, W1e, B1e, W2e, B2e, Wgf, Bgf, Wgb, Bgb,
            Whf, Whb, bhf, bhb, Wof, Wob, bo)


def _k1_body(f_ref, m_ref, b0_ref, w1_ref, b1_ref, w2_ref, b2_ref,
             wgf_ref, bgf_ref, wgb_ref, bgb_ref, gif_ref, gib_ref):
    feat = f_ref[...]
    cls = jnp.clip(feat[:, 0].astype(jnp.int32), 0, 4)
    iota = jax.lax.broadcasted_iota(jnp.int32, (BT1, 5), 1)
    oh = (cls[:, None] == iota).astype(jnp.float32)
    f19 = jnp.concatenate([oh, feat], axis=1)
    h0 = jnp.maximum(
        jnp.dot(f19, m_ref[...], preferred_element_type=jnp.float32)
        + b0_ref[0], 0.0)
    for j in range(NC):
        h0j = h0[:, j * 32:(j + 1) * 32]
        h1 = jnp.maximum(
            jnp.dot(h0j, w1_ref[j], preferred_element_type=jnp.float32)
            + b1_ref[j][None, :], 0.0)
        h2 = jnp.maximum(
            jnp.dot(h1, w2_ref[j], preferred_element_type=jnp.float32)
            + b2_ref[j][None, :], 0.0)
        gf = (jnp.dot(h2, wgf_ref[j], preferred_element_type=jnp.float32)
              + bgf_ref[j][None, :])
        gb = (jnp.dot(h2, wgb_ref[j], preferred_element_type=jnp.float32)
              + bgb_ref[j][None, :])
        gif_ref[:, j, :] = gf
        gib_ref[:, j, :] = gb


def _run_k1(features, M, B0r, W1e, B1e, W2e, B2e, Wgf, Bgf, Wgb, Bgb):
    grid = (NP // NC, B // BT1)
    wspec = lambda bs: pl.BlockSpec(bs, lambda c, b: (c, 0, 0))
    bspec = lambda bs: pl.BlockSpec(bs, lambda c, b: (c, 0))
    return pl.pallas_call(
        _k1_body,
        grid=grid,
        in_specs=[
            pl.BlockSpec((BT1, 14), lambda c, b: (b, 0)),
            pl.BlockSpec((19, NC * 32), lambda c, b: (0, c)),
            pl.BlockSpec((1, 1, NC * 32), lambda c, b: (c, 0, 0)),
            wspec((NC, 32, 64)), bspec((NC, 64)),
            wspec((NC, 64, 32)), bspec((NC, 32)),
            wspec((NC, 32, 48)), bspec((NC, 48)),
            wspec((NC, 32, 48)), bspec((NC, 48)),
        ],
        out_specs=[
            pl.BlockSpec((BT1, NC, 48), lambda c, b: (b, c, 0)),
            pl.BlockSpec((BT1, NC, 48), lambda c, b: (b, c, 0)),
        ],
        out_shape=[
            jax.ShapeDtypeStruct((B, NP, 48), jnp.float32),
            jax.ShapeDtypeStruct((B, NP, 48), jnp.float32),
        ],
        compiler_params=pltpu.CompilerParams(
            dimension_semantics=("parallel", "parallel")),
    )(features, M, B0r, W1e, B1e, W2e, B2e, Wgf, Bgf, Wgb, Bgb)


def _gru_step(gi, gh, h):
    r = jax.nn.sigmoid(gi[0:H] + gh[0:H])
    z = jax.nn.sigmoid(gi[H:2 * H] + gh[H:2 * H])
    c = jnp.tanh(gi[2 * H:] + r * gh[2 * H:])
    return (1.0 - z) * c + z * h


def _k2_body(gitf_ref, gitb_ref, whf_ref, whb_ref, bhf_ref, bhb_ref,
             wof_ref, wob_ref, of_ref, ob_ref, hf_scr, hb_scr,
             hsf_scr, hsb_scr):
    k = pl.program_id(1)

    @pl.when(k == 0)
    def _():
        hf_scr[...] = jnp.zeros_like(hf_scr)
        hb_scr[...] = jnp.zeros_like(hb_scr)

    whf, whb = whf_ref[...], whb_ref[...]
    bhf, bhb = bhf_ref[...], bhb_ref[...]

    def body(s, carry):
        h_f, h_b = carry
        gif = gitf_ref[s]
        ghf = (whf * h_f[None, :, :]).sum(axis=1) + bhf
        h_f = _gru_step(gif, ghf, h_f)
        hsf_scr[s] = h_f
        sb = TT - 1 - s
        gib = gitb_ref[sb]
        ghb = (whb * h_b[None, :, :]).sum(axis=1) + bhb
        h_b = _gru_step(gib, ghb, h_b)
        hsb_scr[sb] = h_b
        return h_f, h_b

    hf, hb = jax.lax.fori_loop(0, TT, body, (hf_scr[...], hb_scr[...]))
    hf_scr[...] = hf
    hb_scr[...] = hb
    of_ref[...] = (hsf_scr[...] * wof_ref[...][None, :, :]).sum(axis=1)
    ob_ref[...] = (hsb_scr[...] * wob_ref[...][None, :, :]).sum(axis=1)


def _run_k2(gitf, gitb, Whf, Whb, bhf, bhb, Wof, Wob):
    nt = B // TT
    grid = (2, nt)
    half3 = lambda: pl.BlockSpec((48, H, NH), lambda c, k: (0, 0, c))
    half2 = lambda a, b_: pl.BlockSpec((a, b_), lambda c, k: (0, c))
    return pl.pallas_call(
        _k2_body,
        grid=grid,
        in_specs=[
            pl.BlockSpec((TT, 48, NH), lambda c, k: (k, 0, c)),
            pl.BlockSpec((TT, 48, NH), lambda c, k: (nt - 1 - k, 0, c)),
            half3(), half3(),
            half2(48, NH), half2(48, NH),
            half2(H, NH), half2(H, NH),
        ],
        out_specs=[
            pl.BlockSpec((TT, NH), lambda c, k: (k, c)),
            pl.BlockSpec((TT, NH), lambda c, k: (nt - 1 - k, c)),
        ],
        out_shape=[
            jax.ShapeDtypeStruct((B, NP), jnp.float32),
            jax.ShapeDtypeStruct((B, NP), jnp.float32),
        ],
        scratch_shapes=[
            pltpu.VMEM((H, NH), jnp.float32),
            pltpu.VMEM((H, NH), jnp.float32),
            pltpu.VMEM((TT, H, NH), jnp.float32),
            pltpu.VMEM((TT, H, NH), jnp.float32),
        ],
        compiler_params=pltpu.CompilerParams(
            dimension_semantics=("parallel", "arbitrary")),
    )(gitf, gitb, Whf, Whb, bhf, bhb, Wof, Wob)


def _k3_body(of_ref, ob_ref, bo_ref, w1_ref, b1_ref, w2_ref, b2_ref,
             dl_ref, res_ref):
    dl = jax.nn.sigmoid(of_ref[...] + ob_ref[...] + bo_ref[...])
    dl_ref[...] = dl
    hid = jnp.maximum(
        jnp.dot(dl, w1_ref[...], preferred_element_type=jnp.float32)
        + b1_ref[...], 0.0)
    res_ref[...] = jax.nn.sigmoid(
        (hid * w2_ref[...]).sum(axis=1, keepdims=True) + b2_ref[...])


def _run_k3(of, ob, bo, W1p, b1, W2r, b2):
    grid = (B // BT1,)
    fixed = lambda bs: pl.BlockSpec(bs, lambda b: tuple(0 for _ in bs))
    return pl.pallas_call(
        _k3_body,
        grid=grid,
        in_specs=[
            pl.BlockSpec((BT1, NP), lambda b: (b, 0)),
            pl.BlockSpec((BT1, NP), lambda b: (b, 0)),
            fixed((1, NP)), fixed((NP, 16)), fixed((1, 16)),
            fixed((1, 16)), fixed((1, 1)),
        ],
        out_specs=[
            pl.BlockSpec((BT1, NP), lambda b: (b, 0)),
            pl.BlockSpec((BT1, 1), lambda b: (b, 0)),
        ],
        out_shape=[
            jax.ShapeDtypeStruct((B, NP), jnp.float32),
            jax.ShapeDtypeStruct((B, 1), jnp.float32),
        ],
        compiler_params=pltpu.CompilerParams(
            dimension_semantics=("parallel",)),
    )(of, ob, bo, W1p, b1, W2r, b2)


@functools.partial(jax.jit, static_argnames=())
def _forward(features, idx7, idx3, p7, p3, out_params):
    (M, B0r, W1e, B1e, W2e, B2e, Wgf, Bgf, Wgb, Bgb,
     Whf, Whb, bhf, bhb, Wof, Wob, bo) = _prep(idx7, idx3, p7, p3)

    gif, gib = _run_k1(features, M, B0r, W1e, B1e, W2e, B2e,
                       Wgf, Bgf, Wgb, Bgb)
    gitf = jnp.transpose(gif, (0, 2, 1))
    gitb = jnp.transpose(gib, (0, 2, 1))
    of, ob = _run_k2(gitf, gitb, Whf, Whb, bhf, bhb, Wof, Wob)

    n = idx7.shape[0] + idx3.shape[0]
    W1p = jnp.zeros((NP, 16), jnp.float32).at[:n].set(out_params['W1'][:n])
    b1 = out_params['b1'].reshape(1, 16)
    W2r = out_params['W2'].reshape(1, 16)
    b2 = out_params['b2'].reshape(1, 1)
    dl, res = _run_k3(of, ob, bo, W1p, b1, W2r, b2)
    return res, dl[:, :n]


def kernel(features, idx7, idx3, p7, p3, out_params):
    return _forward(features, idx7, idx3, p7, p3, out_params)
